# baseline (speedup 1.0000x reference)
import functools

import jax
import jax.numpy as jnp
from jax import lax
from jax.experimental import pallas as pl
from jax.experimental.pallas import tpu as pltpu
from jax.experimental.pallas import tpu_sc as plsc

N = 100000
K = 10000
B = 16384

NC = 2
NS = 16
L = 16
NW = NC * NS
BPW = B // NW
NV = BPW // L


def _body(xt_hbm, i_hbm, dt_hbm, out_hbm,
          idx_v, x_v, d_v, o_v, l_v, w_v,
          sem_i, sem_x, sem_d, sem_o):
    wid = lax.axis_index("s") * NC + lax.axis_index("c")
    base = wid * BPW
    cp_i = pltpu.make_async_copy(i_hbm.at[pl.ds(base, BPW)], idx_v, sem_i)
    cp_i.start()
    cp_x = []
    for c in range(3):
        cp = pltpu.make_async_copy(
            xt_hbm.at[pl.ds(c * B + base, BPW)],
            x_v.at[pl.ds(c * BPW, BPW)], sem_x)
        cp.start()
        cp_x.append(cp)
    cp_d = pltpu.make_async_copy(dt_hbm, d_v, sem_d)
    cp_d.start()

    scale_num = jnp.float32(K - 1)
    scale_den = jnp.float32(N - 1)
    one = jnp.float32(1.0)

    cp_i.wait()

    @plsc.parallel_loop(0, BPW, L, unroll=8)
    def _phase1(s):
        iv = idx_v[pl.ds(s, L)]
        raw = (iv.astype(jnp.float32) * scale_num) / scale_den
        left = raw.astype(jnp.int32)
        leftf = left.astype(jnp.float32)
        exact = raw == leftf
        right = jnp.where(exact, left, left + 1)
        wl = jnp.where(exact, one, raw - leftf)
        l_v[pl.ds(s, L)] = left
        l_v[pl.ds(BPW + s, L)] = right
        w_v[pl.ds(s, L)] = wl

    cp_d.wait()
    for cp in cp_x:
        cp.wait()

    out_copies = []
    for c in range(3):
        coff = c * K
        xoff = c * BPW

        @plsc.parallel_loop(0, BPW, L, unroll=8)
        def _phase2(s):
            wl = w_v[pl.ds(s, L)]
            wr = one - wl
            lf = l_v[pl.ds(s, L)]
            rf = l_v[pl.ds(BPW + s, L)]
            dl = plsc.load_gather(d_v, [lf + coff])
            dr = plsc.load_gather(d_v, [rf + coff])
            xc = x_v[pl.ds(xoff + s, L)]
            o_v[pl.ds(xoff + s, L)] = xc + dl * wl + dr * wr

        cp_o = pltpu.make_async_copy(
            o_v.at[pl.ds(c * BPW, BPW)],
            out_hbm.at[pl.ds(c * B + base, BPW)], sem_o)
        cp_o.start()
        out_copies.append(cp_o)
    for cp_o in out_copies:
        cp_o.wait()


@jax.jit
def kernel(x, i, deltas):
    mesh = plsc.VectorSubcoreMesh(core_axis_name="c", subcore_axis_name="s")
    run = functools.partial(
        pl.kernel,
        mesh=mesh,
        compiler_params=pltpu.CompilerParams(
            needs_layout_passes=False,
            disable_bounds_checks=True,
            skip_device_barrier=True,
        ),
        out_type=jax.ShapeDtypeStruct((3 * B,), jnp.float32),
        scratch_types=[
            pltpu.VMEM((BPW,), jnp.int32),
            pltpu.VMEM((3 * BPW,), jnp.float32),
            pltpu.VMEM((3 * K,), jnp.float32),
            pltpu.VMEM((3 * BPW,), jnp.float32),
            pltpu.VMEM((2 * BPW,), jnp.int32),
            pltpu.VMEM((BPW,), jnp.float32),
            pltpu.SemaphoreType.DMA,
            pltpu.SemaphoreType.DMA,
            pltpu.SemaphoreType.DMA,
            pltpu.SemaphoreType.DMA,
        ],
    )(_body)
    otf = run(x.T.reshape(-1), i, deltas.T.reshape(-1))
    return otf.reshape(3, B).T


# speedup vs baseline: 1.0119x; 1.0119x over previous
import functools

import jax
import jax.numpy as jnp
from jax import lax
from jax.experimental import pallas as pl
from jax.experimental.pallas import tpu as pltpu
from jax.experimental.pallas import tpu_sc as plsc

N = 100000
K = 10000
B = 16384

NC = 2
NS = 16
L = 16
NW = NC * NS
BPW = B // NW
NV = BPW // L


def _body(xt_hbm, i_hbm, dt_hbm, out_hbm,
          idx_v, x_v, d_v, o_v, l_v, w_v,
          sem_i, sem_x, sem_d, sem_o):
    wid = lax.axis_index("s") * NC + lax.axis_index("c")
    base = wid * BPW
    cp_i = pltpu.make_async_copy(i_hbm.at[pl.ds(base, BPW)], idx_v, sem_i)
    cp_i.start()
    cp_x = []
    for c in range(3):
        cp = pltpu.make_async_copy(
            xt_hbm.at[pl.ds(c * B + base, BPW)],
            x_v.at[pl.ds(c * BPW, BPW)], sem_x)
        cp.start()
        cp_x.append(cp)
    cp_d = pltpu.make_async_copy(dt_hbm, d_v, sem_d)
    cp_d.start()

    scale_num = jnp.float32(K - 1)
    scale_den = jnp.float32(N - 1)
    one = jnp.float32(1.0)

    cp_i.wait()

    @plsc.parallel_loop(0, BPW, L, unroll=4)
    def _phase1(s):
        iv = idx_v[pl.ds(s, L)]
        raw = (iv.astype(jnp.float32) * scale_num) / scale_den
        left = raw.astype(jnp.int32)
        leftf = left.astype(jnp.float32)
        exact = raw == leftf
        right = jnp.where(exact, left, left + 1)
        wl = jnp.where(exact, one, raw - leftf)
        l_v[pl.ds(s, L)] = left
        l_v[pl.ds(BPW + s, L)] = right
        w_v[pl.ds(s, L)] = wl

    cp_d.wait()
    for cp in cp_x:
        cp.wait()

    out_copies = []
    for c in range(3):
        coff = c * K
        xoff = c * BPW

        @plsc.parallel_loop(0, BPW, L, unroll=4)
        def _phase2(s):
            wl = w_v[pl.ds(s, L)]
            wr = one - wl
            lf = l_v[pl.ds(s, L)]
            rf = l_v[pl.ds(BPW + s, L)]
            dl = plsc.load_gather(d_v, [lf + coff])
            dr = plsc.load_gather(d_v, [rf + coff])
            xc = x_v[pl.ds(xoff + s, L)]
            o_v[pl.ds(xoff + s, L)] = xc + dl * wl + dr * wr

        cp_o = pltpu.make_async_copy(
            o_v.at[pl.ds(c * BPW, BPW)],
            out_hbm.at[pl.ds(c * B + base, BPW)], sem_o)
        cp_o.start()
        out_copies.append(cp_o)
    for cp_o in out_copies:
        cp_o.wait()


@jax.jit
def kernel(x, i, deltas):
    mesh = plsc.VectorSubcoreMesh(core_axis_name="c", subcore_axis_name="s")
    run = functools.partial(
        pl.kernel,
        mesh=mesh,
        compiler_params=pltpu.CompilerParams(
            needs_layout_passes=False,
            disable_bounds_checks=True,
            skip_device_barrier=True,
        ),
        out_type=jax.ShapeDtypeStruct((3 * B,), jnp.float32),
        scratch_types=[
            pltpu.VMEM((BPW,), jnp.int32),
            pltpu.VMEM((3 * BPW,), jnp.float32),
            pltpu.VMEM((3 * K,), jnp.float32),
            pltpu.VMEM((3 * BPW,), jnp.float32),
            pltpu.VMEM((2 * BPW,), jnp.int32),
            pltpu.VMEM((BPW,), jnp.float32),
            pltpu.SemaphoreType.DMA,
            pltpu.SemaphoreType.DMA,
            pltpu.SemaphoreType.DMA,
            pltpu.SemaphoreType.DMA,
        ],
    )(_body)
    otf = run(x.T.reshape(-1), i, deltas.T.reshape(-1))
    return otf.reshape(3, B).T
